# baseline (device time: 43318 ns/iter reference)
import jax
import jax.numpy as jnp
from jax import lax
from jax.experimental import pallas as pl
from jax.experimental.pallas import tpu as pltpu

N_DEV = 4


def kernel(x, w_mat, scale_x, scale_w):
    m_per, k = x.shape
    _, n = w_mat.shape
    n_per = n // N_DEV

    scale = (scale_x * scale_w).reshape(1, 1)

    def body(x_ref, w_ref, s_ref, out_ref,
             wf_ref, qsend_ref, qrecv_ref, sc_send_ref, sc_recv_ref,
             w_dma_sems, send_sems, sc_send_sems, recv_sems, sc_recv_sems):
        my = lax.axis_index("i")
        s = s_ref[0, 0]

        def w_dma(step):
            jj = lax.rem(my + 1 + step, N_DEV)
            return pltpu.make_async_copy(
                w_ref.at[:, pl.ds(jj * n_per, n_per)],
                wf_ref.at[step % 2],
                w_dma_sems.at[step % 2],
            )

        def quant_send(acc, step):
            jj = lax.rem(my + 1 + step, N_DEV)
            a = jnp.maximum(jnp.max(jnp.abs(acc), axis=0, keepdims=True),
                            1e-20)
            qsend_ref[step] = jnp.round(acc * (127.0 / a)).astype(jnp.int8)
            sc_send_ref[step] = a * (s / 127.0)
            pltpu.make_async_remote_copy(
                src_ref=qsend_ref.at[step],
                dst_ref=qrecv_ref.at[step],
                send_sem=send_sems.at[step],
                recv_sem=recv_sems.at[step],
                device_id=(jj,),
                device_id_type=pl.DeviceIdType.MESH,
            ).start()
            pltpu.make_async_remote_copy(
                src_ref=sc_send_ref.at[step],
                dst_ref=sc_recv_ref.at[step],
                send_sem=sc_send_sems.at[step],
                recv_sem=sc_recv_sems.at[step],
                device_id=(jj,),
                device_id_type=pl.DeviceIdType.MESH,
            ).start()

        w_dma(0).start()

        barrier = pltpu.get_barrier_semaphore()
        for off in range(1, N_DEV):
            pl.semaphore_signal(
                barrier, inc=1,
                device_id=(lax.rem(my + off, N_DEV),),
                device_id_type=pl.DeviceIdType.MESH,
            )
        pl.semaphore_wait(barrier, N_DEV - 1)

        w_dma(0).wait()
        w_dma(1).start()
        acc_prev = jnp.dot(x_ref[...], wf_ref[0],
                           preferred_element_type=jnp.float32)
        for t in range(1, N_DEV):
            w_dma(t).wait()
            if t + 1 < N_DEV:
                w_dma(t + 1).start()
            acc = jnp.dot(x_ref[...], wf_ref[t % 2],
                          preferred_element_type=jnp.float32)
            quant_send(acc_prev, t - 1)
            acc_prev = acc

        out_ref[pl.ds(my * m_per, m_per), :] = jnp.maximum(acc_prev * s, 0.0)

        for t in range(N_DEV - 1):
            wait_d = pltpu.make_async_remote_copy(
                src_ref=qsend_ref.at[t], dst_ref=qrecv_ref.at[t],
                send_sem=send_sems.at[t], recv_sem=recv_sems.at[t],
                device_id=(my,), device_id_type=pl.DeviceIdType.MESH,
            )
            wait_d.wait_recv()
            wait_s = pltpu.make_async_remote_copy(
                src_ref=sc_send_ref.at[t], dst_ref=sc_recv_ref.at[t],
                send_sem=sc_send_sems.at[t], recv_sem=sc_recv_sems.at[t],
                device_id=(my,), device_id_type=pl.DeviceIdType.MESH,
            )
            wait_s.wait_recv()

            src = lax.rem(my + 3 - t, N_DEV)
            y = qrecv_ref[t].astype(jnp.float32) * sc_recv_ref[t]
            out_ref[pl.ds(src * m_per, m_per), :] = jnp.maximum(y, 0.0)

        for t in range(N_DEV - 1):
            data = pltpu.make_async_remote_copy(
                src_ref=qsend_ref.at[t], dst_ref=qrecv_ref.at[t],
                send_sem=send_sems.at[t], recv_sem=recv_sems.at[t],
                device_id=(my,), device_id_type=pl.DeviceIdType.MESH,
            )
            data.wait_send()
            sc = pltpu.make_async_remote_copy(
                src_ref=sc_send_ref.at[t], dst_ref=sc_recv_ref.at[t],
                send_sem=sc_send_sems.at[t], recv_sem=sc_recv_sems.at[t],
                device_id=(my,), device_id_type=pl.DeviceIdType.MESH,
            )
            sc.wait_send()

    return pl.pallas_call(
        body,
        out_shape=jax.ShapeDtypeStruct((N_DEV * m_per, n_per), jnp.float32),
        in_specs=[
            pl.BlockSpec(memory_space=pltpu.VMEM),
            pl.BlockSpec(memory_space=pltpu.MemorySpace.HBM),
            pl.BlockSpec(memory_space=pltpu.SMEM),
        ],
        out_specs=pl.BlockSpec(memory_space=pltpu.VMEM),
        scratch_shapes=[
            pltpu.VMEM((2, k, n_per), jnp.float32),
            pltpu.VMEM((N_DEV - 1, m_per, n_per), jnp.int8),
            pltpu.VMEM((N_DEV - 1, m_per, n_per), jnp.int8),
            pltpu.VMEM((N_DEV - 1, 1, n_per), jnp.float32),
            pltpu.VMEM((N_DEV - 1, 1, n_per), jnp.float32),
            pltpu.SemaphoreType.DMA((2,)),
            pltpu.SemaphoreType.DMA((N_DEV - 1,)),
            pltpu.SemaphoreType.DMA((N_DEV - 1,)),
            pltpu.SemaphoreType.DMA((N_DEV - 1,)),
            pltpu.SemaphoreType.DMA((N_DEV - 1,)),
        ],
        compiler_params=pltpu.CompilerParams(
            collective_id=0,
            vmem_limit_bytes=44 * 1024 * 1024,
        ),
    )(x, w_mat, scale)
